# speedup vs baseline: 12.5803x; 12.5803x over previous
import functools

import jax
import jax.numpy as jnp
from jax import lax
from jax.experimental import pallas as pl
from jax.experimental.pallas import tpu as pltpu
from jax.experimental.pallas import tpu_sc as plsc

NC = 2
NS = 16
LANES = 16

ROW_BLK = 512


def _mesh():
    return plsc.VectorSubcoreMesh(core_axis_name="c", subcore_axis_name="s")


def _make_deg_kernel(E, NPAD, K):
    e_per_tile = E // (NC * NS)
    n_chunks = e_per_tile // K
    per_tile_n = NPAD // NS

    def body(dst_hbm, out_hbm, dst_idx, ones_v, zvec, acc, sem):
        c = lax.axis_index("c")
        s = lax.axis_index("s")
        wid = c * NS + s
        ebase = wid * e_per_tile

        def zb(i, _):
            zvec[pl.ds(i * LANES, LANES)] = jnp.zeros((LANES,), jnp.float32)
            return 0
        lax.fori_loop(0, per_tile_n // LANES, zb, 0)
        pltpu.sync_copy(zvec, acc.at[pl.ds(s * per_tile_n, per_tile_n)])

        def ob(i, _):
            ones_v[pl.ds(i * LANES, LANES)] = jnp.ones((LANES,), jnp.float32)
            return 0
        lax.fori_loop(0, K // LANES, ob, 0)

        plsc.subcore_barrier()

        def chunk(i, _):
            pltpu.sync_copy(dst_hbm.at[pl.ds(ebase + i * K, K)], dst_idx)
            pltpu.sync_copy(ones_v, acc.at[dst_idx], add=True)
            return 0
        lax.fori_loop(0, n_chunks, chunk, 0)

        plsc.subcore_barrier()
        pltpu.sync_copy(acc.at[pl.ds(s * per_tile_n, per_tile_n)],
                        out_hbm.at[c, pl.ds(s * per_tile_n, per_tile_n)])

    return pl.kernel(
        body,
        out_type=jax.ShapeDtypeStruct((NC, NPAD), jnp.float32),
        mesh=_mesh(),
        scratch_types=[
            pltpu.VMEM((K,), jnp.int32),
            pltpu.VMEM((K,), jnp.float32),
            pltpu.VMEM((NPAD // NS,), jnp.float32),
            pltpu.VMEM_SHARED((NPAD,), jnp.float32),
            pltpu.SemaphoreType.DMA,
        ],
    )


def _make_agg_kernel(E, NPAD, D, K):
    e_per_tile = E // (NC * NS)
    n_chunks = e_per_tile // K
    per_tile_n = NPAD // NS
    ZR = 128

    def body(y_hbm, src_hbm, dst_hbm, out_hbm,
             src_idx, dst_idx, rows, zbuf, acc, sem):
        c = lax.axis_index("c")
        s = lax.axis_index("s")
        wid = c * NS + s
        ebase = wid * e_per_tile

        def zb(i, _):
            for j in range(D // LANES):
                zbuf[i, pl.ds(j * LANES, LANES)] = jnp.zeros((LANES,), jnp.float32)
            return 0
        lax.fori_loop(0, ZR, zb, 0)
        for t in range(per_tile_n // ZR):
            pltpu.sync_copy(zbuf, acc.at[pl.ds(s * per_tile_n + t * ZR, ZR)])

        plsc.subcore_barrier()

        def chunk(i, _):
            e0 = ebase + i * K
            pltpu.sync_copy(src_hbm.at[pl.ds(e0, K)], src_idx)
            pltpu.sync_copy(dst_hbm.at[pl.ds(e0, K)], dst_idx)
            pltpu.async_copy(y_hbm.at[src_idx], rows, sem).wait()
            pltpu.sync_copy(rows, acc.at[dst_idx], add=True)
            return 0
        lax.fori_loop(0, n_chunks, chunk, 0)

        plsc.subcore_barrier()
        pltpu.sync_copy(acc.at[pl.ds(s * per_tile_n, per_tile_n)],
                        out_hbm.at[c, pl.ds(s * per_tile_n, per_tile_n)])

    return pl.kernel(
        body,
        out_type=jax.ShapeDtypeStruct((NC, NPAD, D), jnp.float32),
        mesh=_mesh(),
        scratch_types=[
            pltpu.VMEM((K,), jnp.int32),
            pltpu.VMEM((K,), jnp.int32),
            pltpu.VMEM((K, D), jnp.float32),
            pltpu.VMEM((ZR, D), jnp.float32),
            pltpu.VMEM_SHARED((NPAD, D), jnp.float32),
            pltpu.SemaphoreType.DMA,
        ],
    )


def _dinv(d0_ref, d1_ref):
    deg = d0_ref[...] + d1_ref[...] + 1.0
    return 1.0 / jnp.sqrt(deg)


def _tc_first(x_ref, w_ref, d0_ref, d1_ref, y_ref):
    dinv = _dinv(d0_ref, d1_ref)
    y_ref[...] = jnp.dot(x_ref[...], w_ref[...],
                         preferred_element_type=jnp.float32) * dinv


def _lrelu(x):
    return jnp.where(x >= 0, x, 0.01 * x)


def _tc_mid(a0_ref, a1_ref, y_ref, d0_ref, d1_ref, b_ref, w_ref, o_ref):
    dinv = _dinv(d0_ref, d1_ref)
    pre = (a0_ref[...] + a1_ref[...] + y_ref[...]) * dinv + b_ref[...]
    h = _lrelu(pre)
    o_ref[...] = jnp.dot(h, w_ref[...], preferred_element_type=jnp.float32) * dinv


def _tc_last(a0_ref, a1_ref, y_ref, d0_ref, d1_ref, b_ref, w_ref, bf_ref, o_ref):
    dinv = _dinv(d0_ref, d1_ref)
    pre = (a0_ref[...] + a1_ref[...] + y_ref[...]) * dinv + b_ref[...]
    h = _lrelu(pre)
    o_ref[...] = jnp.dot(h, w_ref[...],
                         preferred_element_type=jnp.float32) + bf_ref[...]


def _full_spec(shape):
    return pl.BlockSpec(shape, lambda i: tuple(0 for _ in shape))


def kernel(inputs, edge_index, W1, b1, W2, b2, Wfc, bfc):
    N, D = inputs.shape
    E = edge_index.shape[1]
    NPAD = ((N + ROW_BLK - 1) // ROW_BLK) * ROW_BLK
    K = 80

    src = edge_index[0]
    dst = edge_index[1]

    x = jnp.pad(inputs, ((0, NPAD - N), (0, 0)))
    grid = NPAD // ROW_BLK

    deg_fn = _make_deg_kernel(E, NPAD, K)
    degp = deg_fn(dst)
    d0 = degp[0].reshape(NPAD, 1)
    d1 = degp[1].reshape(NPAD, 1)

    dspec = pl.BlockSpec((ROW_BLK, 1), lambda i: (i, 0))
    rspec = pl.BlockSpec((ROW_BLK, D), lambda i: (i, 0))
    wspec = _full_spec((D, D))
    bspec = _full_spec((1, D))

    y1 = pl.pallas_call(
        _tc_first,
        grid=(grid,),
        in_specs=[rspec, wspec, dspec, dspec],
        out_specs=rspec,
        out_shape=jax.ShapeDtypeStruct((NPAD, D), jnp.float32),
    )(x, W1, d0, d1)

    agg_fn = _make_agg_kernel(E, NPAD, D, K)
    aggp1 = agg_fn(y1, src, dst)

    y2 = pl.pallas_call(
        _tc_mid,
        grid=(grid,),
        in_specs=[rspec, rspec, rspec, dspec, dspec, bspec, wspec],
        out_specs=rspec,
        out_shape=jax.ShapeDtypeStruct((NPAD, D), jnp.float32),
    )(aggp1[0], aggp1[1], y1, d0, d1, b1.reshape(1, D), W2)

    aggp2 = agg_fn(y2, src, dst)

    out = pl.pallas_call(
        _tc_last,
        grid=(grid,),
        in_specs=[rspec, rspec, rspec, dspec, dspec, bspec, wspec, bspec],
        out_specs=rspec,
        out_shape=jax.ShapeDtypeStruct((NPAD, D), jnp.float32),
    )(aggp2[0], aggp2[1], y2, d0, d1, b2.reshape(1, D), Wfc, bfc.reshape(1, D))

    return out[:N]


# speedup vs baseline: 27.3991x; 2.1779x over previous
import functools

import jax
import jax.numpy as jnp
from jax import lax
from jax.experimental import pallas as pl
from jax.experimental.pallas import tpu as pltpu
from jax.experimental.pallas import tpu_sc as plsc

NC = 2
NS = 16
LANES = 16

ROW_BLK = 512


def _mesh():
    return plsc.VectorSubcoreMesh(core_axis_name="c", subcore_axis_name="s")


def _make_deg_kernel(E, NPAD, K):
    e_per_tile = E // (NC * NS)
    n_chunks = e_per_tile // K
    per_tile_n = NPAD // NS

    def body(dst_hbm, out_hbm, dst_buf, ones_v, zvec, acc, sem):
        c = lax.axis_index("c")
        s = lax.axis_index("s")
        wid = c * NS + s

        pltpu.async_copy(dst_hbm.at[wid], dst_buf, sem)

        def zb(i, _):
            zvec[pl.ds(i * LANES, LANES)] = jnp.zeros((LANES,), jnp.float32)
            return 0
        lax.fori_loop(0, per_tile_n // LANES, zb, 0)
        pltpu.sync_copy(zvec, acc.at[pl.ds(s * per_tile_n, per_tile_n)])

        def ob(i, _):
            ones_v[pl.ds(i * LANES, LANES)] = jnp.ones((LANES,), jnp.float32)
            return 0
        lax.fori_loop(0, K // LANES, ob, 0)

        pltpu.make_async_copy(dst_hbm.at[wid], dst_buf, sem).wait()
        plsc.subcore_barrier()

        def chunk(i, _):
            pltpu.sync_copy(ones_v, acc.at[dst_buf.at[i]], add=True)
            return 0
        lax.fori_loop(0, n_chunks, chunk, 0)

        plsc.subcore_barrier()
        pltpu.sync_copy(acc.at[pl.ds(s * per_tile_n, per_tile_n)],
                        out_hbm.at[c, pl.ds(s * per_tile_n, per_tile_n)])

    return pl.kernel(
        body,
        out_type=jax.ShapeDtypeStruct((NC, NPAD), jnp.float32),
        mesh=_mesh(),
        scratch_types=[
            pltpu.VMEM((n_chunks, K), jnp.int32),
            pltpu.VMEM((K,), jnp.float32),
            pltpu.VMEM((NPAD // NS,), jnp.float32),
            pltpu.VMEM_SHARED((NPAD,), jnp.float32),
            pltpu.SemaphoreType.DMA,
        ],
    )


def _make_agg_kernel(E, NPAD, D, K, N_ACC):
    e_per_tile = E // (NC * NS)
    n_chunks = e_per_tile // K
    per_tile_n = N_ACC // NS
    ZR = 8

    def body(y_hbm, src_hbm, dst_hbm, out_hbm,
             src_buf, dst_buf, rows0, rows1, zbuf, acc, isem, sem0, sem1):
        c = lax.axis_index("c")
        s = lax.axis_index("s")
        wid = c * NS + s

        pltpu.async_copy(src_hbm.at[wid], src_buf, isem)
        pltpu.async_copy(dst_hbm.at[wid], dst_buf, isem)

        def zb(i, _):
            for j in range(D // LANES):
                zbuf[i, pl.ds(j * LANES, LANES)] = jnp.zeros((LANES,), jnp.float32)
            return 0
        lax.fori_loop(0, ZR, zb, 0)

        def zc(t, _):
            pltpu.sync_copy(zbuf, acc.at[pl.ds(s * per_tile_n + t * ZR, ZR)])
            return 0
        lax.fori_loop(0, per_tile_n // ZR, zc, 0)

        pltpu.make_async_copy(src_hbm.at[wid], src_buf, isem).wait()
        pltpu.make_async_copy(dst_hbm.at[wid], dst_buf, isem).wait()
        plsc.subcore_barrier()

        def gather(i, rbuf, sem):
            return pltpu.async_copy(
                y_hbm.at[src_buf.at[pl.ds(i * K, K)]], rbuf, sem)

        def gwait(i, rbuf, sem):
            pltpu.make_async_copy(
                y_hbm.at[src_buf.at[pl.ds(i * K, K)]], rbuf, sem).wait()

        def scat(i, rbuf):
            pltpu.sync_copy(rbuf, acc.at[dst_buf.at[i]], add=True)

        gather(0, rows0, sem0)

        def pair(j, _):
            i = 2 * j
            gather(i + 1, rows1, sem1)
            gwait(i, rows0, sem0)
            scat(i, rows0)
            gather(i + 2, rows0, sem0)
            gwait(i + 1, rows1, sem1)
            scat(i + 1, rows1)
            return 0
        lax.fori_loop(0, (n_chunks - 1) // 2, pair, 0)

        gwait(n_chunks - 1, rows0, sem0)
        scat(n_chunks - 1, rows0)

        plsc.subcore_barrier()
        pltpu.sync_copy(acc.at[pl.ds(s * per_tile_n, per_tile_n)],
                        out_hbm.at[c, pl.ds(s * per_tile_n, per_tile_n)])

    return pl.kernel(
        body,
        out_type=jax.ShapeDtypeStruct((NC, NPAD, D), jnp.float32),
        mesh=_mesh(),
        scratch_types=[
            pltpu.VMEM((e_per_tile,), jnp.int32),
            pltpu.VMEM((n_chunks, K), jnp.int32),
            pltpu.VMEM((K, D), jnp.float32),
            pltpu.VMEM((K, D), jnp.float32),
            pltpu.VMEM((ZR, D), jnp.float32),
            pltpu.VMEM_SHARED((N_ACC, D), jnp.float32),
            pltpu.SemaphoreType.DMA,
            pltpu.SemaphoreType.DMA,
            pltpu.SemaphoreType.DMA,
        ],
    )


def _dinv(d0_ref, d1_ref):
    deg = d0_ref[...] + d1_ref[...] + 1.0
    return 1.0 / jnp.sqrt(deg)


def _tc_first(x_ref, w_ref, d0_ref, d1_ref, y_ref):
    dinv = _dinv(d0_ref, d1_ref)
    y_ref[...] = jnp.dot(x_ref[...], w_ref[...],
                         preferred_element_type=jnp.float32) * dinv


def _lrelu(x):
    return jnp.where(x >= 0, x, 0.01 * x)


def _tc_mid(a0_ref, a1_ref, y_ref, d0_ref, d1_ref, b_ref, w_ref, o_ref):
    dinv = _dinv(d0_ref, d1_ref)
    pre = (a0_ref[...] + a1_ref[...] + y_ref[...]) * dinv + b_ref[...]
    h = _lrelu(pre)
    o_ref[...] = jnp.dot(h, w_ref[...], preferred_element_type=jnp.float32) * dinv


def _tc_last(a0_ref, a1_ref, y_ref, d0_ref, d1_ref, b_ref, w_ref, bf_ref, o_ref):
    dinv = _dinv(d0_ref, d1_ref)
    pre = (a0_ref[...] + a1_ref[...] + y_ref[...]) * dinv + b_ref[...]
    h = _lrelu(pre)
    o_ref[...] = jnp.dot(h, w_ref[...],
                         preferred_element_type=jnp.float32) + bf_ref[...]


def _full_spec(shape):
    return pl.BlockSpec(shape, lambda i: tuple(0 for _ in shape))


def kernel(inputs, edge_index, W1, b1, W2, b2, Wfc, bfc):
    N, D = inputs.shape
    E = edge_index.shape[1]
    NPAD = ((N + ROW_BLK - 1) // ROW_BLK) * ROW_BLK

    K = 80
    e_per_tile = E // (NC * NS)
    n_chunks = e_per_tile // K
    src = edge_index[0].reshape(NC * NS, e_per_tile)
    dst = edge_index[1].reshape(NC * NS, n_chunks, K)

    x = jnp.pad(inputs, ((0, NPAD - N), (0, 0)))
    grid = NPAD // ROW_BLK

    deg_fn = _make_deg_kernel(E, NPAD, K)
    degp = deg_fn(dst)
    d0 = degp[0].reshape(NPAD, 1)
    d1 = degp[1].reshape(NPAD, 1)

    dspec = pl.BlockSpec((ROW_BLK, 1), lambda i: (i, 0))
    rspec = pl.BlockSpec((ROW_BLK, D), lambda i: (i, 0))
    wspec = _full_spec((D, D))
    bspec = _full_spec((1, D))

    y1 = pl.pallas_call(
        _tc_first,
        grid=(grid,),
        in_specs=[rspec, wspec, dspec, dspec],
        out_specs=rspec,
        out_shape=jax.ShapeDtypeStruct((NPAD, D), jnp.float32),
    )(x, W1, d0, d1)

    N_ACC = ((N + 15) // 16) * 16
    while (N_ACC // NS) % 8 != 0:
        N_ACC += 16
    agg_fn = _make_agg_kernel(E, NPAD, D, K, N_ACC)
    aggp1 = agg_fn(y1, src, dst)

    y2 = pl.pallas_call(
        _tc_mid,
        grid=(grid,),
        in_specs=[rspec, rspec, rspec, dspec, dspec, bspec, wspec],
        out_specs=rspec,
        out_shape=jax.ShapeDtypeStruct((NPAD, D), jnp.float32),
    )(aggp1[0], aggp1[1], y1, d0, d1, b1.reshape(1, D), W2)

    aggp2 = agg_fn(y2, src, dst)

    out = pl.pallas_call(
        _tc_last,
        grid=(grid,),
        in_specs=[rspec, rspec, rspec, dspec, dspec, bspec, wspec, bspec],
        out_specs=rspec,
        out_shape=jax.ShapeDtypeStruct((NPAD, D), jnp.float32),
    )(aggp2[0], aggp2[1], y2, d0, d1, b2.reshape(1, D), Wfc, bfc.reshape(1, D))

    return out[:N]


# speedup vs baseline: 27.8970x; 1.0182x over previous
import functools

import jax
import jax.numpy as jnp
from jax import lax
from jax.experimental import pallas as pl
from jax.experimental.pallas import tpu as pltpu
from jax.experimental.pallas import tpu_sc as plsc

NC = 2
NS = 16
LANES = 16

ROW_BLK = 512


def _mesh():
    return plsc.VectorSubcoreMesh(core_axis_name="c", subcore_axis_name="s")


def _make_deg_kernel(E, NDEG, K):
    e_per_tile = E // (NC * NS)
    n_chunks = e_per_tile // K
    per_tile_n = NDEG // NS
    zpad = ((per_tile_n + LANES - 1) // LANES) * LANES

    def body(dst_hbm, out_hbm, dst_buf, ones_v, zvec, acc, sem):
        c = lax.axis_index("c")
        s = lax.axis_index("s")
        wid = c * NS + s

        pltpu.async_copy(dst_hbm.at[wid], dst_buf, sem)

        def zb(i, _):
            zvec[pl.ds(i * LANES, LANES)] = jnp.zeros((LANES,), jnp.float32)
            return 0
        lax.fori_loop(0, zpad // LANES, zb, 0)
        pltpu.sync_copy(zvec.at[pl.ds(0, per_tile_n)],
                        acc.at[pl.ds(s * per_tile_n, per_tile_n)])

        def ob(i, _):
            ones_v[pl.ds(i * LANES, LANES)] = jnp.ones((LANES,), jnp.float32)
            return 0
        lax.fori_loop(0, K // LANES, ob, 0)

        pltpu.make_async_copy(dst_hbm.at[wid], dst_buf, sem).wait()
        plsc.subcore_barrier()

        def chunk(i, _):
            pltpu.sync_copy(ones_v, acc.at[dst_buf.at[i]], add=True)
            return 0
        lax.fori_loop(0, n_chunks, chunk, 0)

        plsc.subcore_barrier()
        pltpu.sync_copy(acc.at[pl.ds(s * per_tile_n, per_tile_n)],
                        out_hbm.at[c, pl.ds(s * per_tile_n, per_tile_n)])

    return pl.kernel(
        body,
        out_type=jax.ShapeDtypeStruct((NC, NDEG), jnp.float32),
        mesh=_mesh(),
        scratch_types=[
            pltpu.VMEM((n_chunks, K), jnp.int32),
            pltpu.VMEM((K,), jnp.float32),
            pltpu.VMEM((zpad,), jnp.float32),
            pltpu.VMEM_SHARED((NDEG,), jnp.float32),
            pltpu.SemaphoreType.DMA,
        ],
    )


def _make_agg_kernel(E, D, K, N_ACC):
    e_per_tile = E // (NC * NS)
    n_chunks = e_per_tile // K
    per_tile_n = N_ACC // NS
    ZR = 8

    def body(y_hbm, src_hbm, dst_hbm, out_hbm,
             src_buf, dst_buf, rows0, rows1, zbuf, acc, isem, sem0, sem1):
        c = lax.axis_index("c")
        s = lax.axis_index("s")
        wid = c * NS + s

        pltpu.async_copy(src_hbm.at[wid], src_buf, isem)
        pltpu.async_copy(dst_hbm.at[wid], dst_buf, isem)

        def zb(i, _):
            for j in range(D // LANES):
                zbuf[i, pl.ds(j * LANES, LANES)] = jnp.zeros((LANES,), jnp.float32)
            return 0
        lax.fori_loop(0, ZR, zb, 0)

        def zc(t, _):
            pltpu.sync_copy(zbuf, acc.at[pl.ds(s * per_tile_n + t * ZR, ZR)])
            return 0
        lax.fori_loop(0, per_tile_n // ZR, zc, 0)

        pltpu.make_async_copy(src_hbm.at[wid], src_buf, isem).wait()
        pltpu.make_async_copy(dst_hbm.at[wid], dst_buf, isem).wait()
        plsc.subcore_barrier()

        def gather(i, rbuf, sem):
            return pltpu.async_copy(
                y_hbm.at[src_buf.at[pl.ds(i * K, K)]], rbuf, sem)

        def gwait(i, rbuf, sem):
            pltpu.make_async_copy(
                y_hbm.at[src_buf.at[pl.ds(i * K, K)]], rbuf, sem).wait()

        def scat(i, rbuf):
            pltpu.sync_copy(rbuf, acc.at[dst_buf.at[i]], add=True)

        gather(0, rows0, sem0)

        def pair(j, _):
            i = 2 * j
            gather(i + 1, rows1, sem1)
            gwait(i, rows0, sem0)
            scat(i, rows0)
            gather(i + 2, rows0, sem0)
            gwait(i + 1, rows1, sem1)
            scat(i + 1, rows1)
            return 0
        lax.fori_loop(0, (n_chunks - 1) // 2, pair, 0)

        gwait(n_chunks - 1, rows0, sem0)
        scat(n_chunks - 1, rows0)

        plsc.subcore_barrier()
        pltpu.sync_copy(acc.at[pl.ds(s * per_tile_n, per_tile_n)],
                        out_hbm.at[c, pl.ds(s * per_tile_n, per_tile_n)])

    return pl.kernel(
        body,
        out_type=jax.ShapeDtypeStruct((NC, N_ACC, D), jnp.float32),
        mesh=_mesh(),
        scratch_types=[
            pltpu.VMEM((e_per_tile,), jnp.int32),
            pltpu.VMEM((n_chunks, K), jnp.int32),
            pltpu.VMEM((K, D), jnp.float32),
            pltpu.VMEM((K, D), jnp.float32),
            pltpu.VMEM((ZR, D), jnp.float32),
            pltpu.VMEM_SHARED((N_ACC, D), jnp.float32),
            pltpu.SemaphoreType.DMA,
            pltpu.SemaphoreType.DMA,
            pltpu.SemaphoreType.DMA,
        ],
    )


def _dinv(d0_ref, d1_ref):
    deg = d0_ref[...] + d1_ref[...] + 1.0
    return 1.0 / jnp.sqrt(deg)


def _tc_first(x_ref, w_ref, d0_ref, d1_ref, y_ref):
    dinv = _dinv(d0_ref, d1_ref)
    y_ref[...] = jnp.dot(x_ref[...], w_ref[...],
                         preferred_element_type=jnp.float32) * dinv


def _lrelu(x):
    return jnp.where(x >= 0, x, 0.01 * x)


def _tc_mid(a0_ref, a1_ref, y_ref, d0_ref, d1_ref, b_ref, w_ref, o_ref):
    dinv = _dinv(d0_ref, d1_ref)
    pre = (a0_ref[...] + a1_ref[...] + y_ref[...]) * dinv + b_ref[...]
    h = _lrelu(pre)
    o_ref[...] = jnp.dot(h, w_ref[...], preferred_element_type=jnp.float32) * dinv


def _tc_last(a0_ref, a1_ref, y_ref, d0_ref, d1_ref, b_ref, w_ref, bf_ref, o_ref):
    dinv = _dinv(d0_ref, d1_ref)
    pre = (a0_ref[...] + a1_ref[...] + y_ref[...]) * dinv + b_ref[...]
    h = _lrelu(pre)
    o_ref[...] = jnp.dot(h, w_ref[...],
                         preferred_element_type=jnp.float32) + bf_ref[...]


def _full_spec(shape):
    return pl.BlockSpec(shape, lambda i: tuple(0 for _ in shape))


def kernel(inputs, edge_index, W1, b1, W2, b2, Wfc, bfc):
    N, D = inputs.shape
    E = edge_index.shape[1]

    K = 80
    e_per_tile = E // (NC * NS)
    n_chunks = e_per_tile // K
    src = edge_index[0].reshape(NC * NS, e_per_tile)
    dst = edge_index[1].reshape(NC * NS, n_chunks, K)

    N_ACC = ((N + 15) // 16) * 16
    while (N_ACC // NS) % 8 != 0:
        N_ACC += 16

    grid = N // ROW_BLK

    NDEG = ((N + NS * 128 - 1) // (NS * 128)) * (NS * 128)
    deg_fn = _make_deg_kernel(E, NDEG, K)
    degp = deg_fn(dst)
    d0 = degp[0].reshape(NDEG, 1)
    d1 = degp[1].reshape(NDEG, 1)

    dspec = pl.BlockSpec((ROW_BLK, 1), lambda i: (i, 0))
    rspec = pl.BlockSpec((ROW_BLK, D), lambda i: (i, 0))
    wspec = _full_spec((D, D))
    bspec = _full_spec((1, D))

    y1 = pl.pallas_call(
        _tc_first,
        grid=(grid,),
        in_specs=[rspec, wspec, dspec, dspec],
        out_specs=rspec,
        out_shape=jax.ShapeDtypeStruct((N, D), jnp.float32),
    )(inputs, W1, d0, d1)

    agg_fn = _make_agg_kernel(E, D, K, N_ACC)
    aggp1 = agg_fn(y1, src, dst)

    y2 = pl.pallas_call(
        _tc_mid,
        grid=(grid,),
        in_specs=[rspec, rspec, rspec, dspec, dspec, bspec, wspec],
        out_specs=rspec,
        out_shape=jax.ShapeDtypeStruct((N, D), jnp.float32),
    )(aggp1[0], aggp1[1], y1, d0, d1, b1.reshape(1, D), W2)

    aggp2 = agg_fn(y2, src, dst)

    out = pl.pallas_call(
        _tc_last,
        grid=(grid,),
        in_specs=[rspec, rspec, rspec, dspec, dspec, bspec, wspec, bspec],
        out_specs=rspec,
        out_shape=jax.ShapeDtypeStruct((N, D), jnp.float32),
    )(aggp2[0], aggp2[1], y2, d0, d1, b2.reshape(1, D), Wfc, bfc.reshape(1, D))

    return out
